# Initial kernel scaffold; baseline (speedup 1.0000x reference)
import functools

import jax
import jax.numpy as jnp
import numpy as np
from jax import lax
from jax.experimental import pallas as pl
from jax.experimental.pallas import tpu as pltpu
from jax.experimental.pallas import tpu_sc as plsc

B = 1024
L = 150
D = 25
V = 5001
H = 500
C = 2
DH = 12
DPAD = 16
VP = 5008
NIDX = B * L

NC = 2
NS = 16
NW = NC * NS
B_PER_W = NIDX // NW
CH = 40
CW = 120

_P = np.zeros((D, DPAD), np.float32)
for _j in range(DH):
    _P[2 * _j, _j] = 0.5
    _P[2 * _j + 1, _j] = 0.5


def _precompute(emb_pad, p_mat, w1p, w2, b1r, b2r):
    def body(emb_ref, p_ref, w1_ref, w2_ref, b1_ref, b2_ref,
             pt_ref, wc_ref, bc_ref):
        pt_ref[...] = jnp.dot(emb_ref[...], p_ref[...],
                              preferred_element_type=jnp.float32)
        wc_ref[...] = jnp.dot(w1_ref[...], w2_ref[...],
                              preferred_element_type=jnp.float32)
        bc_ref[...] = jnp.dot(b1_ref[...], w2_ref[...],
                              preferred_element_type=jnp.float32) + b2_ref[...]

    return pl.pallas_call(
        body,
        out_shape=[
            jax.ShapeDtypeStruct((VP, DPAD), jnp.float32),
            jax.ShapeDtypeStruct((L * DPAD, C), jnp.float32),
            jax.ShapeDtypeStruct((1, C), jnp.float32),
        ],
    )(emb_pad, p_mat, w1p, w2, b1r, b2r)


def _sc_gather(ptable, idx3):
    mesh = plsc.VectorSubcoreMesh(core_axis_name="c", subcore_axis_name="s")

    @functools.partial(
        pl.kernel,
        mesh=mesh,
        out_type=jax.ShapeDtypeStruct((NW, CH, CW, DPAD), jnp.float32),
        scratch_types=[
            pltpu.VMEM((CH, CW), jnp.int32),
            pltpu.VMEM((CH, CW, DPAD), jnp.float32),
            pltpu.SemaphoreType.DMA,
        ],
    )
    def k(ptable_hbm, idx_hbm, out_hbm, idx_v, rows_v, sem):
        wid = lax.axis_index("s") * NC + lax.axis_index("c")
        pltpu.sync_copy(idx_hbm.at[wid], idx_v)

        def fire(j, carry):
            pltpu.make_async_copy(
                ptable_hbm.at[idx_v.at[j]], rows_v.at[j], sem).start()
            return carry

        lax.fori_loop(0, CH, fire, 0)

        def drain(j, carry):
            pltpu.make_async_copy(
                ptable_hbm.at[idx_v.at[j]], rows_v.at[j], sem).wait()
            return carry

        lax.fori_loop(0, CH, drain, 0)
        pltpu.sync_copy(rows_v, out_hbm.at[wid])

    return k(ptable, idx3)


def _final_matmul(x, wc, bc):
    def body(x_ref, w_ref, b_ref, o_ref):
        o_ref[...] = jnp.dot(x_ref[...], w_ref[...],
                             preferred_element_type=jnp.float32) + b_ref[...]

    return pl.pallas_call(
        body,
        out_shape=jax.ShapeDtypeStruct((B, C), jnp.float32),
    )(x, wc, bc)


def kernel(sents, _, emb_table, W1, b1, W2, b2):
    emb_pad = jnp.pad(emb_table, ((0, VP - V), (0, 0)))
    w1p = jnp.pad(W1.reshape(L, DH, H), ((0, 0), (0, DPAD - DH), (0, 0)))
    w1p = w1p.reshape(L * DPAD, H)
    ptable, wc, bc = _precompute(
        emb_pad, jnp.asarray(_P), w1p, W2,
        b1.reshape(1, H), b2.reshape(1, C))
    idx3 = sents.astype(jnp.int32).reshape(NW, CH, CW)
    x16 = _sc_gather(ptable, idx3)
    return _final_matmul(x16.reshape(B, L * DPAD), wc, bc)


# speedup vs baseline: 9.0424x; 9.0424x over previous
import functools

import jax
import jax.numpy as jnp
import numpy as np
from jax import lax
from jax.experimental import pallas as pl
from jax.experimental.pallas import tpu as pltpu
from jax.experimental.pallas import tpu_sc as plsc

B = 1024
L = 150
D = 25
V = 5001
H = 500
C = 2
DH = 12
DPAD = 16
VP = 5008
NIDX = B * L

NC = 2
NS = 16
NW = NC * NS
B_PER_W = NIDX // NW
CH = 40
CW = 120

_P = np.zeros((D, DPAD), np.float32)
for _j in range(DH):
    _P[2 * _j, _j] = 0.5
    _P[2 * _j + 1, _j] = 0.5


def _precompute(emb_pad, p_mat, w1p, w2, b1r, b2r):
    def body(emb_ref, p_ref, w1_ref, w2_ref, b1_ref, b2_ref,
             pt_ref, wc_ref, bc_ref):
        pt_ref[...] = jnp.dot(emb_ref[...], p_ref[...],
                              preferred_element_type=jnp.float32)
        wc_ref[...] = jnp.dot(w1_ref[...], w2_ref[...],
                              preferred_element_type=jnp.float32)
        bc_ref[...] = jnp.dot(b1_ref[...], w2_ref[...],
                              preferred_element_type=jnp.float32) + b2_ref[...]

    return pl.pallas_call(
        body,
        out_shape=[
            jax.ShapeDtypeStruct((VP, DPAD), jnp.float32),
            jax.ShapeDtypeStruct((L * DPAD, C), jnp.float32),
            jax.ShapeDtypeStruct((1, C), jnp.float32),
        ],
    )(emb_pad, p_mat, w1p, w2, b1r, b2r)


def _sc_gather(ptable, idx3):
    mesh = plsc.VectorSubcoreMesh(core_axis_name="c", subcore_axis_name="s")

    @functools.partial(
        pl.kernel,
        mesh=mesh,
        out_type=jax.ShapeDtypeStruct((NW, CH, CW, DPAD), jnp.float32),
        scratch_types=[
            pltpu.VMEM((CH, CW), jnp.int32),
            pltpu.VMEM((CH, CW, DPAD), jnp.float32),
            pltpu.SemaphoreType.DMA,
        ],
        compiler_params=pltpu.CompilerParams(use_tc_tiling_on_sc=False),
    )
    def k(ptable_hbm, idx_hbm, out_hbm, idx_v, rows_v, sem):
        wid = lax.axis_index("s") * NC + lax.axis_index("c")
        pltpu.sync_copy(idx_hbm.at[wid], idx_v)

        def fire(j, carry):
            pltpu.make_async_copy(
                ptable_hbm.at[idx_v.at[j]], rows_v.at[j], sem).start()
            return carry

        lax.fori_loop(0, CH, fire, 0)

        def drain(j, carry):
            pltpu.make_async_copy(
                ptable_hbm.at[idx_v.at[j]], rows_v.at[j], sem).wait()
            return carry

        lax.fori_loop(0, CH, drain, 0)
        pltpu.sync_copy(rows_v, out_hbm.at[wid])

    return k(ptable, idx3)


def _final_matmul(x, wc, bc):
    def body(x_ref, w_ref, b_ref, o_ref):
        o_ref[...] = jnp.dot(x_ref[...], w_ref[...],
                             preferred_element_type=jnp.float32) + b_ref[...]

    return pl.pallas_call(
        body,
        out_shape=jax.ShapeDtypeStruct((B, C), jnp.float32),
    )(x, wc, bc)


def kernel(sents, _, emb_table, W1, b1, W2, b2):
    emb_pad = jnp.pad(emb_table, ((0, VP - V), (0, 0)))
    w1p = jnp.pad(W1.reshape(L, DH, H), ((0, 0), (0, DPAD - DH), (0, 0)))
    w1p = w1p.reshape(L * DPAD, H)
    ptable, wc, bc = _precompute(
        emb_pad, jnp.asarray(_P), w1p, W2,
        b1.reshape(1, H), b2.reshape(1, C))
    idx3 = sents.astype(jnp.int32).reshape(NW, CH, CW)
    x16 = _sc_gather(ptable, idx3)
    return _final_matmul(x16.reshape(B, L * DPAD), wc, bc)


# speedup vs baseline: 10.6287x; 1.1754x over previous
import functools

import jax
import jax.numpy as jnp
import numpy as np
from jax import lax
from jax.experimental import pallas as pl
from jax.experimental.pallas import tpu as pltpu
from jax.experimental.pallas import tpu_sc as plsc

B = 1024
L = 150
D = 25
V = 5001
H = 500
C = 2
DH = 12
DPAD = 16
NIDX = B * L

NC = 2
NS = 16
NW = NC * NS
B_PER_W = NIDX // NW
CH = 40
CW = 120

_P = np.zeros((D, DPAD), np.float32)
for _j in range(DH):
    _P[2 * _j, _j] = 0.5
    _P[2 * _j + 1, _j] = 0.5


def _precompute(emb, p_mat, w1, w2, b1r, b2r):
    def body(emb_ref, p_ref, w1_ref, w2_ref, b1_ref, b2_ref,
             pt_ref, wc_ref, bc_ref):
        pt_ref[...] = jnp.dot(emb_ref[...], p_ref[...],
                              preferred_element_type=jnp.float32)
        wc_ref[...] = jnp.dot(w1_ref[...], w2_ref[...],
                              preferred_element_type=jnp.float32)
        bc_ref[...] = jnp.dot(b1_ref[...], w2_ref[...],
                              preferred_element_type=jnp.float32) + b2_ref[...]

    return pl.pallas_call(
        body,
        out_shape=[
            jax.ShapeDtypeStruct((V, DPAD), jnp.float32),
            jax.ShapeDtypeStruct((L * DH, C), jnp.float32),
            jax.ShapeDtypeStruct((1, C), jnp.float32),
        ],
    )(emb, p_mat, w1, w2, b1r, b2r)


def _sc_gather(ptable, idx3):
    mesh = plsc.VectorSubcoreMesh(core_axis_name="c", subcore_axis_name="s")

    @functools.partial(
        pl.kernel,
        mesh=mesh,
        out_type=jax.ShapeDtypeStruct((NW, CH, CW, DPAD), jnp.float32),
        scratch_types=[
            pltpu.VMEM((CH, CW), jnp.int32),
            pltpu.VMEM((CH, CW, DPAD), jnp.float32),
            pltpu.SemaphoreType.DMA,
        ],
        compiler_params=pltpu.CompilerParams(use_tc_tiling_on_sc=False),
    )
    def k(ptable_hbm, idx_hbm, out_hbm, idx_v, rows_v, sem):
        wid = lax.axis_index("s") * NC + lax.axis_index("c")
        pltpu.sync_copy(idx_hbm.at[wid], idx_v)

        def fire(j, carry):
            pltpu.make_async_copy(
                ptable_hbm.at[idx_v.at[j]], rows_v.at[j], sem).start()
            return carry

        lax.fori_loop(0, CH, fire, 0)

        def drain(j, carry):
            pltpu.make_async_copy(
                ptable_hbm.at[idx_v.at[j]], rows_v.at[j], sem).wait()
            return carry

        lax.fori_loop(0, CH, drain, 0)
        pltpu.sync_copy(rows_v, out_hbm.at[wid])

    return k(ptable, idx3)


def _final_matmul(x, wc, bc):
    def body(x_ref, w_ref, b_ref, o_ref):
        o_ref[...] = jnp.dot(x_ref[...], w_ref[...],
                             preferred_element_type=jnp.float32) + b_ref[...]

    return pl.pallas_call(
        body,
        out_shape=jax.ShapeDtypeStruct((B, C), jnp.float32),
    )(x, wc, bc)


def kernel(sents, _, emb_table, W1, b1, W2, b2):
    ptable, wc12, bc = _precompute(
        emb_table, jnp.asarray(_P), W1, W2,
        b1.reshape(1, H), b2.reshape(1, C))
    wc = jnp.pad(wc12.reshape(L, DH, C), ((0, 0), (0, DPAD - DH), (0, 0)))
    wc = wc.reshape(L * DPAD, C)
    idx3 = sents.astype(jnp.int32).reshape(NW, CH, CW)
    x16 = _sc_gather(ptable, idx3)
    return _final_matmul(x16.reshape(B, L * DPAD), wc, bc)


# speedup vs baseline: 13.0574x; 1.2285x over previous
import functools

import jax
import jax.numpy as jnp
import numpy as np
from jax import lax
from jax.experimental import pallas as pl
from jax.experimental.pallas import tpu as pltpu
from jax.experimental.pallas import tpu_sc as plsc

B = 1024
L = 150
D = 25
V = 5001
H = 500
C = 2
DH = 12
DPAD = 16
NIDX = B * L

NC = 2
NS = 16
NW = NC * NS
B_PER_W = NIDX // NW
CH = 40
CW = 120
BPW = B // NW
GRP = 8
NG = BPW // GRP

_P = np.zeros((D, DPAD), np.float32)
for _j in range(DH):
    _P[2 * _j, _j] = 0.5
    _P[2 * _j + 1, _j] = 0.5


def _precompute(emb, p_mat, w1, w2, b1r, b2r):
    def body(emb_ref, p_ref, w1_ref, w2_ref, b1_ref, b2_ref,
             pt_ref, wc_ref, bc_ref):
        pt_ref[...] = jnp.dot(emb_ref[...], p_ref[...],
                              preferred_element_type=jnp.float32)
        wc_ref[...] = jnp.dot(w1_ref[...], w2_ref[...],
                              preferred_element_type=jnp.float32)
        bc_ref[...] = jnp.dot(b1_ref[...], w2_ref[...],
                              preferred_element_type=jnp.float32) + b2_ref[...]

    return pl.pallas_call(
        body,
        out_shape=[
            jax.ShapeDtypeStruct((V, DPAD), jnp.float32),
            jax.ShapeDtypeStruct((L * DH, C), jnp.float32),
            jax.ShapeDtypeStruct((1, C), jnp.float32),
        ],
    )(emb, p_mat, w1, w2, b1r, b2r)


def _sc_fused(ptable, idx3, wc_sc, bcv):
    mesh = plsc.VectorSubcoreMesh(core_axis_name="c", subcore_axis_name="s")

    @functools.partial(
        pl.kernel,
        mesh=mesh,
        out_type=jax.ShapeDtypeStruct((NW, BPW * C), jnp.float32),
        scratch_types=[
            pltpu.VMEM((CH, CW), jnp.int32),
            pltpu.VMEM((B_PER_W, DPAD), jnp.float32),
            pltpu.VMEM((L, C, DPAD), jnp.float32),
            pltpu.VMEM((C, DPAD), jnp.float32),
            pltpu.VMEM((BPW * C,), jnp.float32),
            pltpu.SemaphoreType.DMA,
        ],
        compiler_params=pltpu.CompilerParams(
            use_tc_tiling_on_sc=False, needs_layout_passes=False),
    )
    def k(pt_hbm, idx_hbm, wc_hbm, bcv_hbm, out_hbm,
          idx_v, rows_v, wc_v, bcv_v, out_v, sem):
        wid = lax.axis_index("s") * NC + lax.axis_index("c")
        pltpu.sync_copy(idx_hbm.at[wid], idx_v)
        pltpu.sync_copy(wc_hbm, wc_v)
        pltpu.sync_copy(bcv_hbm, bcv_v)

        def fire(j, carry):
            pltpu.make_async_copy(
                pt_hbm.at[idx_v.at[j]],
                rows_v.at[pl.ds(j * CW, CW)], sem).start()
            return carry

        lax.fori_loop(0, CH, fire, 0)

        def drain(j, carry):
            pltpu.make_async_copy(
                pt_hbm.at[idx_v.at[j]],
                rows_v.at[pl.ds(j * CW, CW)], sem).wait()
            return carry

        lax.fori_loop(0, CH, drain, 0)

        lanes = lax.iota(jnp.int32, 16)

        def group_body(g, carry):
            def l_body(l, accs):
                w0 = wc_v[l, 0]
                w1 = wc_v[l, 1]
                base = g * (GRP * L) + l
                new = []
                for kk in range(GRP):
                    row = rows_v[base + kk * L]
                    new.append(accs[2 * kk] + row * w0)
                    new.append(accs[2 * kk + 1] + row * w1)
                return tuple(new)

            init = tuple(bcv_v[kk % 2] for kk in range(2 * GRP))
            accs = lax.fori_loop(0, L, l_body, init)
            out_vec = jnp.zeros((16,), jnp.float32)
            for kk in range(GRP):
                s0 = jnp.sum(accs[2 * kk])
                s1 = jnp.sum(accs[2 * kk + 1])
                out_vec = jnp.where(lanes == 2 * kk, s0, out_vec)
                out_vec = jnp.where(lanes == 2 * kk + 1, s1, out_vec)
            out_v[pl.ds(g * 16, 16)] = out_vec
            return carry

        lax.fori_loop(0, NG, group_body, 0)
        pltpu.sync_copy(out_v, out_hbm.at[wid])

    return k(ptable, idx3, wc_sc, bcv)


def kernel(sents, _, emb_table, W1, b1, W2, b2):
    ptable, wc12, bc = _precompute(
        emb_table, jnp.asarray(_P), W1, W2,
        b1.reshape(1, H), b2.reshape(1, C))
    wc_sc = jnp.pad(wc12.reshape(L, DH, C).transpose(0, 2, 1),
                    ((0, 0), (0, 0), (0, DPAD - DH)))
    bcv = jnp.pad(bc.reshape(C, 1), ((0, 0), (0, DPAD - 1)))
    idx3 = sents.astype(jnp.int32).reshape(NW, CH, CW)
    out = _sc_fused(ptable, idx3, wc_sc, bcv)
    return out.reshape(B, C)
